# baseline (speedup 1.0000x reference)
import functools
import jax
import jax.numpy as jnp
from jax import lax
from jax.experimental import pallas as pl
from jax.experimental.pallas import tpu as pltpu
from jax.experimental.pallas import tpu_sc as plsc

_B, _H, _V, _D = 8, 16, 8192, 64
_N = 4096
_NC, _NS = 2, 16
_NW = _NC * _NS
_PAIRS = _B * _H
_PPW = _PAIRS // _NW
_CH = 128
_NCH = _N // _CH


def _sc_gather(t3, idx2):
    mesh = plsc.VectorSubcoreMesh(core_axis_name="c", subcore_axis_name="s")

    @functools.partial(
        pl.kernel,
        out_type=jax.ShapeDtypeStruct((_PAIRS, _N, _D), jnp.float32),
        mesh=mesh,
        compiler_params=pltpu.CompilerParams(use_tc_tiling_on_sc=False),
        scratch_types=[
            pltpu.VMEM((_NCH, _CH), jnp.int32),
            pltpu.VMEM((_CH, _D), jnp.float32),
            pltpu.SemaphoreType.DMA,
        ],
    )
    def body(t_hbm, idx_hbm, out_hbm, idx_v, rows_v, sem):
        cid = lax.axis_index("c")
        sid = lax.axis_index("s")
        wid = sid * _NC + cid
        pltpu.sync_copy(idx_hbm, idx_v)

        def pair_loop(q, carry):
            p = wid * _PPW + q

            def chunk_loop(c, carry2):
                pltpu.async_copy(
                    t_hbm.at[p].at[idx_v.at[c]], rows_v, sem
                ).wait()
                pltpu.sync_copy(rows_v, out_hbm.at[p].at[pl.ds(c * _CH, _CH)])
                return carry2

            return lax.fori_loop(0, _NCH, chunk_loop, carry)

        lax.fori_loop(0, _PPW, pair_loop, 0)

    return body(t3, idx2)


def kernel(t, idx):
    t3 = t.reshape(_PAIRS, _V, _D)
    idx2 = idx.astype(jnp.int32).reshape(_NCH, _CH)
    out = _sc_gather(t3, idx2)
    return out.reshape(_B, _H, _N, _D)


# speedup vs baseline: 1.0946x; 1.0946x over previous
import functools
import jax
import jax.numpy as jnp
from jax import lax
from jax.experimental import pallas as pl
from jax.experimental.pallas import tpu as pltpu
from jax.experimental.pallas import tpu_sc as plsc

_B, _H, _V, _D = 8, 16, 8192, 64
_N = 4096
_NC, _NS = 2, 16
_NW = _NC * _NS
_PAIRS = _B * _H
_PPW = _PAIRS // _NW
_CH = 128
_NCH = _N // _CH
_GPB = 4
_BR = _GPB * _CH
_NB = _PPW * _N // _BR


def _sc_gather(t3, idx2):
    mesh = plsc.VectorSubcoreMesh(core_axis_name="c", subcore_axis_name="s")

    @functools.partial(
        pl.kernel,
        out_type=jax.ShapeDtypeStruct((_PAIRS, _N, _D), jnp.float32),
        mesh=mesh,
        compiler_params=pltpu.CompilerParams(use_tc_tiling_on_sc=False),
        scratch_types=[
            pltpu.VMEM((_NCH, _CH), jnp.int32),
            pltpu.VMEM((_BR, _D), jnp.float32),
            pltpu.VMEM((_BR, _D), jnp.float32),
            pltpu.SemaphoreType.DMA,
            pltpu.SemaphoreType.DMA,
            pltpu.SemaphoreType.DMA,
            pltpu.SemaphoreType.DMA,
        ],
    )
    def body(t_hbm, idx_hbm, out_hbm, idx_v, buf0, buf1, gs0, gs1, ss0, ss1):
        cid = lax.axis_index("c")
        sid = lax.axis_index("s")
        wid = sid * _NC + cid
        pltpu.sync_copy(idx_hbm, idx_v)

        def issue_gathers(b, buf, sem):
            q = b // (_N // _BR)
            r = b % (_N // _BR)
            p = wid * _PPW + q
            for j in range(_GPB):
                pltpu.async_copy(
                    t_hbm.at[p].at[idx_v.at[r * _GPB + j]],
                    buf.at[pl.ds(j * _CH, _CH)],
                    sem,
                )

        def wait_block(buf, sem):
            pltpu.make_async_copy(
                out_hbm.at[0].at[pl.ds(0, _BR)], buf, sem
            ).wait()

        def issue_store(b, buf, sem):
            q = b // (_N // _BR)
            r = b % (_N // _BR)
            p = wid * _PPW + q
            pltpu.async_copy(buf, out_hbm.at[p].at[pl.ds(r * _BR, _BR)], sem)

        def wait_store(buf, sem):
            pltpu.make_async_copy(
                buf, out_hbm.at[0].at[pl.ds(0, _BR)], sem
            ).wait()

        issue_gathers(0, buf0, gs0)
        issue_gathers(1, buf1, gs1)
        wait_block(buf0, gs0)
        issue_store(0, buf0, ss0)

        def loop(i, carry):
            b0 = 2 * i
            b1 = 2 * i + 1
            wait_store(buf0, ss0)
            issue_gathers(b0, buf0, gs0)
            wait_block(buf1, gs1)
            issue_store(b0 - 1, buf1, ss1)
            wait_store(buf1, ss1)
            issue_gathers(b1, buf1, gs1)
            wait_block(buf0, gs0)
            issue_store(b0, buf0, ss0)
            return carry

        lax.fori_loop(1, _NB // 2, loop, 0)

        wait_store(buf0, ss0)
        wait_block(buf1, gs1)
        issue_store(_NB - 1, buf1, ss1)
        wait_store(buf1, ss1)

    return body(t3, idx2)


def kernel(t, idx):
    t3 = t.reshape(_PAIRS, _V, _D)
    idx2 = idx.astype(jnp.int32).reshape(_NCH, _CH)
    out = _sc_gather(t3, idx2)
    return out.reshape(_B, _H, _N, _D)
